# baseline (speedup 1.0000x reference)
import functools

import jax
import jax.numpy as jnp
import numpy as np
from jax import lax
from jax.experimental import pallas as pl
from jax.experimental.pallas import tpu as pltpu
from jax.experimental.pallas import tpu_sc as plsc

VOCAB = 30522
EMB = 128
EMBW = EMB // 2
HID = 512
B = 128
S = 512
EPS = 1e-12

NC = 2
NS = 16
NW = NC * NS

N_ROWS = B * S
ROWS_PER_W = N_ROWS // NW
CHUNK = 512
N_CHUNKS = ROWS_PER_W // CHUNK

_PERM = np.concatenate(
    [128 * c + np.concatenate([np.arange(64) * 2, np.arange(64) * 2 + 1])
     for c in range(3)]
)


def _pack_table(word_emb):
    tv = jax.lax.bitcast_convert_type(word_emb, jnp.int32)
    ev, ov = tv[:, 0::2], tv[:, 1::2]

    def rnd16(u):
        return ((u + 0x7FFF + ((u >> 16) & 1)) >> 16) & 0xFFFF

    return rnd16(ev) | (rnd16(ov) << 16)


def _sc_gather(table_hbm, idx_hbm, out_hbm, idx_v, rows_v, gsem0, gsem1, ssem0, ssem1):
    wid = lax.axis_index("s") * NC + lax.axis_index("c")
    base = wid * ROWS_PER_W
    pltpu.sync_copy(idx_hbm.at[pl.ds(base, ROWS_PER_W)], idx_v)
    gsems = (gsem0, gsem1)
    ssems = (ssem0, ssem1)

    def gather_start(j, bb):
        return pltpu.async_copy(
            table_hbm.at[idx_v.at[pl.ds(j * CHUNK, CHUNK)]], rows_v.at[bb], gsems[bb]
        )

    g = [gather_start(0, 0), None]
    scat = [None, None]
    for j in range(N_CHUNKS):
        b = j & 1
        if j + 1 < N_CHUNKS:
            if scat[1 - b] is not None:
                scat[1 - b].wait()
            g[1 - b] = gather_start(j + 1, 1 - b)
        g[b].wait()
        scat[b] = pltpu.async_copy(
            rows_v.at[b], out_hbm.at[pl.ds(base + j * CHUNK, CHUNK)], ssems[b]
        )
    for b in (0, 1):
        if scat[b] is not None:
            scat[b].wait()


def _gather_rows(table_packed, ids):
    gather = functools.partial(
        pl.kernel,
        out_type=jax.ShapeDtypeStruct((N_ROWS, EMBW), jnp.int32),
        mesh=plsc.VectorSubcoreMesh(
            core_axis_name="c", subcore_axis_name="s", num_cores=NC
        ),
        compiler_params=pltpu.CompilerParams(use_tc_tiling_on_sc=False),
        scratch_types=[
            pltpu.VMEM((ROWS_PER_W,), jnp.int32),
            pltpu.VMEM((2, CHUNK, EMBW), jnp.int32),
            pltpu.SemaphoreType.DMA,
            pltpu.SemaphoreType.DMA,
            pltpu.SemaphoreType.DMA,
            pltpu.SemaphoreType.DMA,
        ],
    )(_sc_gather)
    return gather(table_packed, ids)


BG = 8


def _tc_dense(e_ref, tt_ref, posb_ref, te_ref, gam_ref, bet_ref, w_ref, out_ref):
    u = e_ref[...]
    lo = pltpu.bitcast(u << 16, jnp.float32)
    hi = pltpu.bitcast(u & jnp.int32(-65536), jnp.float32)
    zg = jnp.zeros((BG, 1, EMBW), jnp.float32)
    l_lo = jnp.concatenate([lo[:, 1:, :], zg], axis=1)
    l_hi = jnp.concatenate([hi[:, 1:, :], zg], axis=1)
    r_lo = jnp.concatenate([zg, lo[:, :-1, :]], axis=1)
    r_hi = jnp.concatenate([zg, hi[:, :-1, :]], axis=1)
    tri = jnp.concatenate([l_lo, l_hi, lo, hi, r_lo, r_hi], axis=2)
    tri = tri.reshape(BG * S, 3 * EMB)
    x = jnp.dot(tri, w_ref[...], preferred_element_type=jnp.float32)
    x = x.reshape(BG, S, HID)
    te = te_ref[...]
    tt = tt_ref[...]
    typ = te[0][None, None, :] + tt[:, :, None] * (te[1] - te[0])[None, None, :]
    emb = x + posb_ref[...][None, :, :] + typ
    mean = jnp.mean(emb, axis=-1, keepdims=True)
    cen = emb - mean
    var = jnp.mean(cen * cen, axis=-1, keepdims=True)
    norm = cen * lax.rsqrt(var + EPS)
    out_ref[...] = norm * gam_ref[...][0][None, None, :] + bet_ref[...][0][None, None, :]


def kernel(input_ids, token_type_ids, word_emb, pos_emb, type_emb, W, b, gamma, beta):
    ids = input_ids.reshape(-1).astype(jnp.int32)
    e = _gather_rows(_pack_table(word_emb), ids).reshape(B, S, EMBW)

    tt_f = token_type_ids.astype(jnp.float32)
    posb = pos_emb + b[None, :]
    gam = gamma.reshape(1, HID)
    bet = beta.reshape(1, HID)
    w_perm = W[_PERM, :]

    grid = (B // BG,)
    out = pl.pallas_call(
        _tc_dense,
        grid=grid,
        in_specs=[
            pl.BlockSpec((BG, S, EMBW), lambda i: (i, 0, 0)),
            pl.BlockSpec((BG, S), lambda i: (i, 0)),
            pl.BlockSpec((S, HID), lambda i: (0, 0)),
            pl.BlockSpec((2, HID), lambda i: (0, 0)),
            pl.BlockSpec((1, HID), lambda i: (0, 0)),
            pl.BlockSpec((1, HID), lambda i: (0, 0)),
            pl.BlockSpec((3 * EMB, HID), lambda i: (0, 0)),
        ],
        out_specs=pl.BlockSpec((BG, S, HID), lambda i: (i, 0, 0)),
        out_shape=jax.ShapeDtypeStruct((B, S, HID), jnp.float32),
    )(e, tt_f, posb, type_emb, gam, bet, w_perm)
    return out


# speedup vs baseline: 3.8934x; 3.8934x over previous
import functools

import jax
import jax.numpy as jnp
import numpy as np
from jax import lax
from jax.experimental import pallas as pl
from jax.experimental.pallas import tpu as pltpu
from jax.experimental.pallas import tpu_sc as plsc

VOCAB = 30522
EMB = 128
EMBW = EMB // 2
HID = 512
B = 128
S = 512
EPS = 1e-12

NC = 2
NS = 16
NW = NC * NS

N_ROWS = B * S
ROWS_PER_W = N_ROWS // NW
CHUNK = 512
N_CHUNKS = ROWS_PER_W // CHUNK

def _pack_table(word_emb):
    tv = jax.lax.bitcast_convert_type(word_emb, jnp.int32)
    lo, hi = tv[:, :EMBW], tv[:, EMBW:]

    def rnd16(u):
        return ((u + 0x7FFF + ((u >> 16) & 1)) >> 16) & 0xFFFF

    return rnd16(lo) | (rnd16(hi) << 16)


def _sc_gather(table_hbm, idx_hbm, out_hbm, idx_v, rows_v, gsem0, gsem1, ssem0, ssem1):
    wid = lax.axis_index("s") * NC + lax.axis_index("c")
    base = wid * ROWS_PER_W
    pltpu.sync_copy(idx_hbm.at[pl.ds(base, ROWS_PER_W)], idx_v)
    gsems = (gsem0, gsem1)
    ssems = (ssem0, ssem1)

    def gather_start(j, bb):
        return pltpu.async_copy(
            table_hbm.at[idx_v.at[pl.ds(j * CHUNK, CHUNK)]], rows_v.at[bb], gsems[bb]
        )

    g = [gather_start(0, 0), None]
    scat = [None, None]
    for j in range(N_CHUNKS):
        b = j & 1
        if j + 1 < N_CHUNKS:
            if scat[1 - b] is not None:
                scat[1 - b].wait()
            g[1 - b] = gather_start(j + 1, 1 - b)
        g[b].wait()
        scat[b] = pltpu.async_copy(
            rows_v.at[b], out_hbm.at[pl.ds(base + j * CHUNK, CHUNK)], ssems[b]
        )
    for b in (0, 1):
        if scat[b] is not None:
            scat[b].wait()


def _gather_rows(table_packed, ids):
    gather = functools.partial(
        pl.kernel,
        out_type=jax.ShapeDtypeStruct((N_ROWS, EMBW), jnp.int32),
        mesh=plsc.VectorSubcoreMesh(
            core_axis_name="c", subcore_axis_name="s", num_cores=NC
        ),
        compiler_params=pltpu.CompilerParams(use_tc_tiling_on_sc=False),
        scratch_types=[
            pltpu.VMEM((ROWS_PER_W,), jnp.int32),
            pltpu.VMEM((2, CHUNK, EMBW), jnp.int32),
            pltpu.SemaphoreType.DMA,
            pltpu.SemaphoreType.DMA,
            pltpu.SemaphoreType.DMA,
            pltpu.SemaphoreType.DMA,
        ],
    )(_sc_gather)
    return gather(table_packed, ids)


BG = 8


def _tc_dense(e_ref, tt_ref, posb_ref, te_ref, gam_ref, bet_ref, w_ref, out_ref):
    u = e_ref[...]
    lo = pltpu.bitcast(u << 16, jnp.float32)
    hi = pltpu.bitcast(u & jnp.int32(-65536), jnp.float32)
    zg = jnp.zeros((BG, 1, EMBW), jnp.float32)
    l_lo = jnp.concatenate([lo[:, 1:, :], zg], axis=1)
    l_hi = jnp.concatenate([hi[:, 1:, :], zg], axis=1)
    r_lo = jnp.concatenate([zg, lo[:, :-1, :]], axis=1)
    r_hi = jnp.concatenate([zg, hi[:, :-1, :]], axis=1)
    tri = jnp.concatenate([l_lo, l_hi, lo, hi, r_lo, r_hi], axis=2)
    tri = tri.reshape(BG * S, 3 * EMB)
    x = jnp.dot(tri, w_ref[...], preferred_element_type=jnp.float32)
    x = x.reshape(BG, S, HID)
    te = te_ref[...]
    tt = tt_ref[...]
    typ = te[0][None, None, :] + tt[:, :, None] * (te[1] - te[0])[None, None, :]
    emb = x + posb_ref[...][None, :, :] + typ
    mean = jnp.mean(emb, axis=-1, keepdims=True)
    cen = emb - mean
    var = jnp.mean(cen * cen, axis=-1, keepdims=True)
    norm = cen * lax.rsqrt(var + EPS)
    out_ref[...] = norm * gam_ref[...][0][None, None, :] + bet_ref[...][0][None, None, :]


def kernel(input_ids, token_type_ids, word_emb, pos_emb, type_emb, W, b, gamma, beta):
    ids = input_ids.reshape(-1).astype(jnp.int32)
    e = _gather_rows(_pack_table(word_emb), ids).reshape(B, S, EMBW)

    tt_f = token_type_ids.astype(jnp.float32)
    posb = pos_emb + b[None, :]
    gam = gamma.reshape(1, HID)
    bet = beta.reshape(1, HID)
    w_perm = W

    grid = (B // BG,)
    out = pl.pallas_call(
        _tc_dense,
        grid=grid,
        in_specs=[
            pl.BlockSpec((BG, S, EMBW), lambda i: (i, 0, 0)),
            pl.BlockSpec((BG, S), lambda i: (i, 0)),
            pl.BlockSpec((S, HID), lambda i: (0, 0)),
            pl.BlockSpec((2, HID), lambda i: (0, 0)),
            pl.BlockSpec((1, HID), lambda i: (0, 0)),
            pl.BlockSpec((1, HID), lambda i: (0, 0)),
            pl.BlockSpec((3 * EMB, HID), lambda i: (0, 0)),
        ],
        out_specs=pl.BlockSpec((BG, S, HID), lambda i: (i, 0, 0)),
        out_shape=jax.ShapeDtypeStruct((B, S, HID), jnp.float32),
    )(e, tt_f, posb, type_emb, gam, bet, w_perm)
    return out
